# initial kernel scaffold (unmeasured)
import jax
import jax.numpy as jnp
from jax import lax
from jax.experimental import pallas as pl
from jax.experimental.pallas import tpu as pltpu

N_DEV = 32


def kernel(x, w_mat, scale_x, scale_w):
    m_per, k = x.shape
    _, n = w_mat.shape
    n_per = n // N_DEV

    def body(x_ref, w_ref, sx_ref, sw_ref, out_ref, comm_ref, send_sems, recv_sems):
        my = lax.axis_index("i")
        scale = sx_ref[0] * sw_ref[0]
        xb = x_ref[:, :].astype(jnp.bfloat16)

        for s in range(N_DEV):
            t = lax.rem(my + s, N_DEV)
            wblk = w_ref[:, pl.ds(t * n_per, n_per)].astype(jnp.bfloat16)
            blk = lax.dot_general(
                xb, wblk, (((1,), (0,)), ((), ())),
                preferred_element_type=jnp.float32,
            ) * scale
            if s == 0:
                out_ref[pl.ds(my * m_per, m_per), :] = blk
            else:
                comm_ref[s, :, :] = blk
                rdma = pltpu.make_async_remote_copy(
                    src_ref=comm_ref.at[s],
                    dst_ref=out_ref.at[pl.ds(my * m_per, m_per), :],
                    send_sem=send_sems.at[s],
                    recv_sem=recv_sems.at[s],
                    device_id=(t,),
                    device_id_type=pl.DeviceIdType.MESH,
                )
                rdma.start()

        for s in range(1, N_DEV):
            src = lax.rem(my - s + N_DEV, N_DEV)
            recv = pltpu.make_async_remote_copy(
                src_ref=comm_ref.at[s],
                dst_ref=out_ref.at[pl.ds(src * m_per, m_per), :],
                send_sem=send_sems.at[s],
                recv_sem=recv_sems.at[s],
                device_id=(src,),
                device_id_type=pl.DeviceIdType.MESH,
            )
            recv.wait_recv()

        for s in range(1, N_DEV):
            t = lax.rem(my + s, N_DEV)
            send = pltpu.make_async_remote_copy(
                src_ref=comm_ref.at[s],
                dst_ref=out_ref.at[pl.ds(my * m_per, m_per), :],
                send_sem=send_sems.at[s],
                recv_sem=recv_sems.at[s],
                device_id=(t,),
                device_id_type=pl.DeviceIdType.MESH,
            )
            send.wait_send()

    return pl.pallas_call(
        body,
        out_shape=jax.ShapeDtypeStruct((N_DEV * m_per, n_per), jnp.float32),
        in_specs=[
            pl.BlockSpec(memory_space=pltpu.VMEM),
            pl.BlockSpec(memory_space=pltpu.VMEM),
            pl.BlockSpec(memory_space=pltpu.SMEM),
            pl.BlockSpec(memory_space=pltpu.SMEM),
        ],
        out_specs=pl.BlockSpec(memory_space=pltpu.VMEM),
        scratch_shapes=[
            pltpu.VMEM((N_DEV, m_per, n_per), jnp.float32),
            pltpu.SemaphoreType.DMA((N_DEV,)),
            pltpu.SemaphoreType.DMA((N_DEV,)),
        ],
        compiler_params=pltpu.CompilerParams(collective_id=0),
    )(x, w_mat, scale_x, scale_w)


# baseline (device time: 74962 ns/iter reference)
import jax
import jax.numpy as jnp
from jax import lax
from jax.experimental import pallas as pl
from jax.experimental.pallas import tpu as pltpu

N_DEV = 32


def kernel(x, w_mat, scale_x, scale_w):
    m_per, k = x.shape
    _, n = w_mat.shape
    n_per = n // N_DEV

    def body(x_ref, w_ref, sx_ref, sw_ref, out_ref,
             xb_ref, wbuf_ref, comm_ref, copy_sems, send_sems, recv_sems):
        my = lax.axis_index("i")
        scale = sx_ref[0] * sw_ref[0]
        xb_ref[:, :] = x_ref[:, :].astype(jnp.bfloat16)

        def wcopy(s):
            t = lax.rem(my + s, N_DEV)
            return pltpu.make_async_copy(
                w_ref.at[:, pl.ds(t * n_per, n_per)],
                wbuf_ref.at[s % 2],
                copy_sems.at[s % 2],
            )

        wcopy(0).start()
        for s in range(N_DEV):
            t = lax.rem(my + s, N_DEV)
            if s + 1 < N_DEV:
                wcopy(s + 1).start()
            wcopy(s).wait()
            blk = lax.dot_general(
                xb_ref[:, :], wbuf_ref[s % 2].astype(jnp.bfloat16),
                (((1,), (0,)), ((), ())),
                preferred_element_type=jnp.float32,
            ) * scale
            if s == 0:
                out_ref[pl.ds(my * m_per, m_per), :] = blk
            else:
                comm_ref[s, :, :] = blk
                rdma = pltpu.make_async_remote_copy(
                    src_ref=comm_ref.at[s],
                    dst_ref=out_ref.at[pl.ds(my * m_per, m_per), :],
                    send_sem=send_sems.at[s],
                    recv_sem=recv_sems.at[s],
                    device_id=(t,),
                    device_id_type=pl.DeviceIdType.MESH,
                )
                rdma.start()

        for s in range(1, N_DEV):
            src = lax.rem(my - s + N_DEV, N_DEV)
            recv = pltpu.make_async_remote_copy(
                src_ref=comm_ref.at[s],
                dst_ref=out_ref.at[pl.ds(src * m_per, m_per), :],
                send_sem=send_sems.at[s],
                recv_sem=recv_sems.at[s],
                device_id=(src,),
                device_id_type=pl.DeviceIdType.MESH,
            )
            recv.wait_recv()

        for s in range(1, N_DEV):
            t = lax.rem(my + s, N_DEV)
            send = pltpu.make_async_remote_copy(
                src_ref=comm_ref.at[s],
                dst_ref=out_ref.at[pl.ds(my * m_per, m_per), :],
                send_sem=send_sems.at[s],
                recv_sem=recv_sems.at[s],
                device_id=(t,),
                device_id_type=pl.DeviceIdType.MESH,
            )
            send.wait_send()

    return pl.pallas_call(
        body,
        out_shape=jax.ShapeDtypeStruct((N_DEV * m_per, n_per), jnp.float32),
        in_specs=[
            pl.BlockSpec(memory_space=pltpu.VMEM),
            pl.BlockSpec(memory_space=pl.ANY),
            pl.BlockSpec(memory_space=pltpu.SMEM),
            pl.BlockSpec(memory_space=pltpu.SMEM),
        ],
        out_specs=pl.BlockSpec(memory_space=pltpu.VMEM),
        scratch_shapes=[
            pltpu.VMEM((m_per, k), jnp.bfloat16),
            pltpu.VMEM((2, k, n_per), jnp.float32),
            pltpu.VMEM((N_DEV, m_per, n_per), jnp.float32),
            pltpu.SemaphoreType.DMA((2,)),
            pltpu.SemaphoreType.DMA((N_DEV,)),
            pltpu.SemaphoreType.DMA((N_DEV,)),
        ],
    )(x, w_mat, scale_x, scale_w)
